# Initial kernel scaffold; baseline (speedup 1.0000x reference)
import jax
import jax.numpy as jnp
from jax.experimental import pallas as pl


def kernel(x, edge_index, edge_attr, params):
    raise NotImplementedError("write your pallas kernel here")


# speedup vs baseline: 22.4532x; 22.4532x over previous
import jax
import jax.numpy as jnp
from jax import lax
from jax.experimental import pallas as pl

_NEG = -1e30


def _leaky(v):
    return jnp.where(v > 0, v, 0.2 * v)


def _dot(a, b, dims):
    return lax.dot_general(a, b, (dims, ((), ())),
                           preferred_element_type=jnp.float32,
                           precision=lax.Precision.HIGHEST)


def _mm(a, b):
    return _dot(a, b, ((1,), (0,)))


def _as_row(col, eye):
    return _dot(col, eye, ((0,), (0,)))


def _seg_softmax_ne(logit_row, gdst_b):
    s = jnp.where(gdst_b, logit_row, _NEG)
    m = jnp.max(s, axis=1, keepdims=True)
    ex = jnp.where(gdst_b, jnp.exp(s - m), 0.0)
    den = jnp.sum(ex, axis=1, keepdims=True) + 1e-16
    return ex / den


def kernel(x, edge_index, edge_attr, params):
    p = params
    N = x.shape[0]
    E = edge_index.shape[1]
    CT_N = p["ca_W3"].shape[1]
    H = p["lstm_Whh"].shape[1]
    OUT = p["gin_W2_3"].shape[1]

    args = [
        x,
        edge_index,
        edge_attr,
        p["lstm_Wih"],
        (p["lstm_bih"] + p["lstm_bhh"])[None, :],
        p["g2_Wl"],
        p["g2_Wr"],
        p["g2_att"][None, :],
    ]
    for li in range(4):
        args += [p["ca_W%d" % li], p["ca_asrc%d" % li][None, :],
                 p["ca_adst%d" % li][None, :], p["ca_b%d" % li][None, :]]
    for li in range(2):
        args += [p["ct_W%d" % li], p["ct_asrc%d" % li][None, :],
                 p["ct_adst%d" % li][None, :], p["ct_b%d" % li][None, :]]
    for li in range(4):
        args += [p["gin_W1_%d" % li], p["gin_b1_%d" % li][None, :],
                 p["gin_W2_%d" % li], p["gin_b2_%d" % li][None, :]]

    def fused(*refs):
        out_ref = refs[-1]
        xv = refs[0][...]
        ei = refs[1][...]
        ea = refs[2][...]

        src_row = ei[0:1, :]
        dst_row = ei[1:2, :]

        def eye(n):
            return (lax.broadcasted_iota(jnp.int32, (n, n), 0)
                    == lax.broadcasted_iota(jnp.int32, (n, n), 1)
                    ).astype(jnp.float32)

        eye_e, eye_ct = eye(E), eye(CT_N)
        node5 = lax.broadcasted_iota(jnp.int32, (N, E), 0)
        node32 = lax.broadcasted_iota(jnp.int32, (CT_N, E), 0)
        gs5_b = node5 == src_row
        gd5_b = node5 == dst_row
        gs5 = gs5_b.astype(jnp.float32)
        gd5 = gd5_b.astype(jnp.float32)
        gs32 = (node32 == src_row).astype(jnp.float32)
        gd32 = (node32 == dst_row).astype(jnp.float32)

        gates = _dot(xv, refs[3][...], ((1,), (1,))) + refs[4][...]
        c = jax.nn.sigmoid(gates[:, 0:H]) * jnp.tanh(gates[:, 2 * H:3 * H])
        h = jax.nn.sigmoid(gates[:, 3 * H:4 * H]) * jnp.tanh(c)

        xl = _mm(xv, refs[5][...])
        xr = _mm(xv, refs[6][...])
        feat = _leaky(_dot(gs5, xl, ((0,), (0,))) + _dot(gd5, xr, ((0,), (0,))))
        logit = jnp.sum(feat * refs[7][...], axis=1, keepdims=True)
        a2 = _seg_softmax_ne(_as_row(logit, eye_e), gd5_b)
        ew_row = _as_row(ea, eye_e) * jnp.sum(a2, axis=0, keepdims=True)

        z = h
        for li in range(4):
            w, asrc, adst, b = refs[8 + 4 * li: 12 + 4 * li]
            hw = _mm(z, w[...])
            s_src = jnp.sum(hw * asrc[...], axis=1, keepdims=True)
            s_dst = jnp.sum(hw * adst[...], axis=1, keepdims=True)
            e_col = _leaky(_dot(gs5, s_src, ((0,), (0,)))
                           + _dot(gd5, s_dst, ((0,), (0,))))
            alpha = _seg_softmax_ne(_as_row(e_col, eye_e), gd5_b) * ew_row
            z = _dot(alpha, _dot(gs5, hw, ((0,), (0,))), ((1,), (0,))) + b[...]
            if li < 3:
                z = jnp.maximum(z, 0.0)

        t = _dot(z, eye(N), ((0,), (0,)))

        ct_offdiag = (lax.broadcasted_iota(jnp.int32, (CT_N, CT_N), 0)
                      != lax.broadcasted_iota(jnp.int32, (CT_N, CT_N), 1))
        for li in range(2):
            w, asrc, adst, b = refs[24 + 4 * li: 28 + 4 * li]
            hw = _mm(t, w[...])
            s_src = jnp.sum(hw * asrc[...], axis=1, keepdims=True)
            s_dst = jnp.sum(hw * adst[...], axis=1, keepdims=True)
            emat = _leaky(s_src + _as_row(s_dst, eye_ct))
            emat = jnp.where(ct_offdiag, emat, _NEG)
            m = jnp.max(emat, axis=0, keepdims=True)
            ex = jnp.where(ct_offdiag, jnp.exp(emat - m), 0.0)
            alpha = ex / (jnp.sum(ex, axis=0, keepdims=True) + 1e-16)
            t = _dot(alpha, hw, ((0,), (0,))) + b[...]
            if li < 1:
                t = jnp.maximum(t, 0.0)

        y = _dot(t, eye_ct, ((0,), (0,)))
        adj = _dot(gd32, gs32, ((1,), (1,)))
        for li in range(4):
            w1, b1, w2, b2 = refs[32 + 4 * li: 36 + 4 * li]
            hg = y + _mm(adj, y)
            hg = jnp.maximum(_mm(hg, w1[...]) + b1[...], 0.0)
            y = _mm(hg, w2[...]) + b2[...]
            if li < 3:
                y = jnp.maximum(y, 0.0)

        out_ref[...] = y

    return pl.pallas_call(
        fused,
        out_shape=jax.ShapeDtypeStruct((CT_N, OUT), jnp.float32),
    )(*args)


# speedup vs baseline: 24.0300x; 1.0702x over previous
import jax
import jax.numpy as jnp
from jax import lax
from jax.experimental import pallas as pl

_NEG = -1e30


def _leaky(v):
    return jnp.where(v > 0, v, 0.2 * v)


def _dot(a, b, dims):
    return lax.dot_general(a, b, (dims, ((), ())),
                           preferred_element_type=jnp.float32,
                           precision=lax.Precision.HIGHEST)


def _mm(a, b):
    return _dot(a, b, ((1,), (0,)))


def _as_row(col, eye):
    return _dot(col, eye, ((0,), (0,)))


def _seg_softmax_ne(logit_row, gdst_b):
    s = jnp.where(gdst_b, logit_row, _NEG)
    m = jnp.max(s, axis=1, keepdims=True)
    ex = jnp.where(gdst_b, jnp.exp(s - m), 0.0)
    den = jnp.sum(ex, axis=1, keepdims=True) + 1e-16
    return ex / den


def kernel(x, edge_index, edge_attr, params):
    p = params
    N = x.shape[0]
    E = edge_index.shape[1]
    CT_N = p["ca_W3"].shape[1]
    H = p["lstm_Whh"].shape[1]
    OUT = p["gin_W2_3"].shape[1]

    args = [
        x,
        edge_index,
        edge_attr,
        p["lstm_Wih"],
        p["lstm_bih"][None, :],
        p["lstm_bhh"][None, :],
        p["g2_Wl"],
        p["g2_Wr"],
        p["g2_att"][None, :],
    ]
    for li in range(4):
        args += [p["ca_W%d" % li], p["ca_asrc%d" % li][None, :],
                 p["ca_adst%d" % li][None, :], p["ca_b%d" % li][None, :]]
    for li in range(2):
        args += [p["ct_W%d" % li], p["ct_asrc%d" % li][None, :],
                 p["ct_adst%d" % li][None, :], p["ct_b%d" % li][None, :]]
    for li in range(4):
        args += [p["gin_W1_%d" % li], p["gin_b1_%d" % li][None, :],
                 p["gin_W2_%d" % li], p["gin_b2_%d" % li][None, :]]

    def fused(*refs):
        out_ref = refs[-1]
        xv = refs[0][...]
        ei = refs[1][...]
        ea = refs[2][...]

        src_row = ei[0:1, :]
        dst_row = ei[1:2, :]

        def eye(n):
            return (lax.broadcasted_iota(jnp.int32, (n, n), 0)
                    == lax.broadcasted_iota(jnp.int32, (n, n), 1)
                    ).astype(jnp.float32)

        eye_e, eye_ct = eye(E), eye(CT_N)
        node5 = lax.broadcasted_iota(jnp.int32, (N, E), 0)
        node32 = lax.broadcasted_iota(jnp.int32, (CT_N, E), 0)
        gs5_b = node5 == src_row
        gd5_b = node5 == dst_row
        gs5 = gs5_b.astype(jnp.float32)
        gd5 = gd5_b.astype(jnp.float32)
        gs32 = (node32 == src_row).astype(jnp.float32)
        gd32 = (node32 == dst_row).astype(jnp.float32)

        gates = (_dot(xv, refs[3][...], ((1,), (1,)))
                 + refs[4][...] + refs[5][...])
        c = jax.nn.sigmoid(gates[:, 0:H]) * jnp.tanh(gates[:, 2 * H:3 * H])
        h = jax.nn.sigmoid(gates[:, 3 * H:4 * H]) * jnp.tanh(c)

        xl = _mm(xv, refs[6][...])
        xr = _mm(xv, refs[7][...])
        feat = _leaky(_dot(gs5, xl, ((0,), (0,))) + _dot(gd5, xr, ((0,), (0,))))
        logit = jnp.sum(feat * refs[8][...], axis=1, keepdims=True)
        a2 = _seg_softmax_ne(_as_row(logit, eye_e), gd5_b)
        ew_row = _as_row(ea, eye_e) * jnp.sum(a2, axis=0, keepdims=True)

        z = h
        for li in range(4):
            w, asrc, adst, b = refs[9 + 4 * li: 13 + 4 * li]
            hw = _mm(z, w[...])
            s_src = jnp.sum(hw * asrc[...], axis=1, keepdims=True)
            s_dst = jnp.sum(hw * adst[...], axis=1, keepdims=True)
            e_col = _leaky(_dot(gs5, s_src, ((0,), (0,)))
                           + _dot(gd5, s_dst, ((0,), (0,))))
            alpha = _seg_softmax_ne(_as_row(e_col, eye_e), gd5_b) * ew_row
            z = _dot(alpha, _dot(gs5, hw, ((0,), (0,))), ((1,), (0,))) + b[...]
            if li < 3:
                z = jnp.maximum(z, 0.0)

        t = _dot(z, eye(N), ((0,), (0,)))

        ct_offdiag = (lax.broadcasted_iota(jnp.int32, (CT_N, CT_N), 0)
                      != lax.broadcasted_iota(jnp.int32, (CT_N, CT_N), 1))
        for li in range(2):
            w, asrc, adst, b = refs[25 + 4 * li: 29 + 4 * li]
            hw = _mm(t, w[...])
            s_src = jnp.sum(hw * asrc[...], axis=1, keepdims=True)
            s_dst = jnp.sum(hw * adst[...], axis=1, keepdims=True)
            emat = _leaky(s_src + _as_row(s_dst, eye_ct))
            emat = jnp.where(ct_offdiag, emat, _NEG)
            m = jnp.max(emat, axis=0, keepdims=True)
            ex = jnp.where(ct_offdiag, jnp.exp(emat - m), 0.0)
            alpha = ex / (jnp.sum(ex, axis=0, keepdims=True) + 1e-16)
            t = _dot(alpha, hw, ((0,), (0,))) + b[...]
            if li < 1:
                t = jnp.maximum(t, 0.0)

        y = _dot(t, eye_ct, ((0,), (0,)))
        adj = _dot(gd32, gs32, ((1,), (1,)))
        for li in range(4):
            w1, b1, w2, b2 = refs[33 + 4 * li: 37 + 4 * li]
            hg = y + _mm(adj, y)
            hg = jnp.maximum(_mm(hg, w1[...]) + b1[...], 0.0)
            y = _mm(hg, w2[...]) + b2[...]
            if li < 3:
                y = jnp.maximum(y, 0.0)

        out_ref[...] = y

    return pl.pallas_call(
        fused,
        out_shape=jax.ShapeDtypeStruct((CT_N, OUT), jnp.float32),
    )(*args)


# speedup vs baseline: 24.7910x; 1.0317x over previous
import jax
import jax.numpy as jnp
from jax import lax
from jax.experimental import pallas as pl

_NEG = -1e30


def _leaky(v):
    return jnp.where(v > 0, v, 0.2 * v)


def _dot(a, b, dims):
    return lax.dot_general(a, b, (dims, ((), ())),
                           preferred_element_type=jnp.float32,
                           precision=lax.Precision.HIGH)


def _mm(a, b):
    return _dot(a, b, ((1,), (0,)))


def _seg_softmax_ne(logit_row, gdst_b):
    s = jnp.where(gdst_b, logit_row, _NEG)
    m = jnp.max(s, axis=1, keepdims=True)
    ex = jnp.where(gdst_b, jnp.exp(s - m), 0.0)
    den = jnp.sum(ex, axis=1, keepdims=True) + 1e-16
    return ex / den


def kernel(x, edge_index, edge_attr, params):
    p = params
    N = x.shape[0]
    E = edge_index.shape[1]
    CT_N = p["ca_W3"].shape[1]
    H = p["lstm_Whh"].shape[1]
    OUT = p["gin_W2_3"].shape[1]

    args = [
        x,
        edge_index,
        edge_attr[:, 0][None, :],
        p["lstm_Wih"],
        p["lstm_bih"][None, :],
        p["lstm_bhh"][None, :],
        p["g2_Wl"],
        p["g2_Wr"],
        p["g2_att"][:, None],
    ]
    for li in range(4):
        b = p["ca_b%d" % li]
        args += [p["ca_W%d" % li], p["ca_asrc%d" % li][None, :],
                 p["ca_adst%d" % li][None, :],
                 b[:, None] if li == 3 else b[None, :]]
    for li in range(2):
        b = p["ct_b%d" % li]
        args += [p["ct_W%d" % li], p["ct_asrc%d" % li][:, None],
                 p["ct_adst%d" % li][None, :],
                 b[:, None] if li == 1 else b[None, :]]
    for li in range(4):
        args += [p["gin_W1_%d" % li], p["gin_b1_%d" % li][None, :],
                 p["gin_W2_%d" % li], p["gin_b2_%d" % li][None, :]]

    def fused(*refs):
        out_ref = refs[-1]
        xv = refs[0][...]
        ei = refs[1][...]
        ea_row = refs[2][...]

        src_row = ei[0:1, :]
        dst_row = ei[1:2, :]

        node5 = lax.broadcasted_iota(jnp.int32, (N, E), 0)
        node32 = lax.broadcasted_iota(jnp.int32, (CT_N, E), 0)
        gs5_b = node5 == src_row
        gd5_b = node5 == dst_row
        gs5 = gs5_b.astype(jnp.float32)
        gd5 = gd5_b.astype(jnp.float32)
        gs32 = (node32 == src_row).astype(jnp.float32)
        gd32 = (node32 == dst_row).astype(jnp.float32)

        gates = (_dot(xv, refs[3][...], ((1,), (1,)))
                 + refs[4][...] + refs[5][...])
        c = jax.nn.sigmoid(gates[:, 0:H]) * jnp.tanh(gates[:, 2 * H:3 * H])
        h = jax.nn.sigmoid(gates[:, 3 * H:4 * H]) * jnp.tanh(c)

        xlT = _dot(refs[6][...], xv, ((0,), (1,)))
        xrT = _dot(refs[7][...], xv, ((0,), (1,)))
        featT = _leaky(_mm(xlT, gs5) + _mm(xrT, gd5))
        logit_row = jnp.sum(featT * refs[8][...], axis=0, keepdims=True)
        a2 = _seg_softmax_ne(logit_row, gd5_b)
        ew_row = ea_row * jnp.sum(a2, axis=0, keepdims=True)

        z = h
        for li in range(4):
            w, asrc, adst, b = refs[9 + 4 * li: 13 + 4 * li]
            hw = _mm(z, w[...])
            s_src = jnp.sum(hw * asrc[...], axis=1, keepdims=True)
            s_dst = jnp.sum(hw * adst[...], axis=1, keepdims=True)
            e_row = _leaky(jnp.sum(gs5 * s_src, axis=0, keepdims=True)
                           + jnp.sum(gd5 * s_dst, axis=0, keepdims=True))
            alpha = _seg_softmax_ne(e_row, gd5_b) * ew_row
            msgs = _dot(gs5, hw, ((0,), (0,)))
            if li < 3:
                z = jnp.maximum(_mm(alpha, msgs) + b[...], 0.0)
            else:
                z = _dot(msgs, alpha, ((0,), (1,))) + b[...]

        offdiag = (lax.broadcasted_iota(jnp.int32, (CT_N, CT_N), 0)
                   != lax.broadcasted_iota(jnp.int32, (CT_N, CT_N), 1))
        t = z
        for li in range(2):
            w, asrc, adst, b = refs[25 + 4 * li: 29 + 4 * li]
            hw = _mm(t, w[...])
            hwT = _dot(w[...], t, ((0,), (1,)))
            s_src_row = jnp.sum(hwT * asrc[...], axis=0, keepdims=True)
            s_dst_col = jnp.sum(hw * adst[...], axis=1, keepdims=True)
            ematT = _leaky(s_dst_col + s_src_row)
            ematT = jnp.where(offdiag, ematT, _NEG)
            m = jnp.max(ematT, axis=1, keepdims=True)
            ex = jnp.where(offdiag, jnp.exp(ematT - m), 0.0)
            alphaT = ex / (jnp.sum(ex, axis=1, keepdims=True) + 1e-16)
            if li < 1:
                t = jnp.maximum(_mm(alphaT, hw) + b[...], 0.0)
            else:
                y = _dot(hw, alphaT, ((0,), (1,))) + b[...]

        adj = _dot(gd32, gs32, ((1,), (1,)))
        for li in range(4):
            w1, b1, w2, b2 = refs[33 + 4 * li: 37 + 4 * li]
            hg = y + _mm(adj, y)
            hg = jnp.maximum(_mm(hg, w1[...]) + b1[...], 0.0)
            y = _mm(hg, w2[...]) + b2[...]
            if li < 3:
                y = jnp.maximum(y, 0.0)

        out_ref[...] = y

    return pl.pallas_call(
        fused,
        out_shape=jax.ShapeDtypeStruct((CT_N, OUT), jnp.float32),
    )(*args)
